# baseline (speedup 1.0000x reference)
import jax
import jax.numpy as jnp
from jax.experimental import pallas as pl
from jax.experimental.pallas import tpu as pltpu

IN_DIMS = 1024
HID = 1024
OUT = 1024
B = 4096

_R = 1024
_NB = B // _R


def _block_stats(h_block):
    hp = h_block.astype(jnp.float32)
    return (jnp.sum(hp, axis=0, keepdims=True),
            jnp.sum(hp * hp, axis=0, keepdims=True))


def _body(x_ref, ws_ref, bs_ref, g0_ref, b0_ref, we_ref, be_ref,
          out_ref, h_s, acc_s, w2_s, b2_s):
    p = pl.program_id(0)
    i = pl.program_id(1)

    @pl.when(p == 0)
    def _phase0():
        h = jnp.dot(x_ref[:], ws_ref[:],
                    preferred_element_type=jnp.float32)
        h = jnp.maximum(h + bs_ref[:], 0.0)
        h_s[pl.ds(i * _R, _R), :] = h.astype(jnp.bfloat16)

        @pl.when(i == 0)
        def _init():
            acc_s[0:1, :] = jnp.zeros((1, HID), jnp.float32)
            acc_s[1:2, :] = jnp.zeros((1, HID), jnp.float32)

        @pl.when(i > 0)
        def _accum():
            cs, cq = _block_stats(h_s[pl.ds((i - 1) * _R, _R), :])
            acc_s[0:1, :] = acc_s[0:1, :] + cs
            acc_s[1:2, :] = acc_s[1:2, :] + cq

    @pl.when(p == 1)
    def _phase1():
        @pl.when(i == 0)
        def _fold_bn():
            cs, cq = _block_stats(h_s[pl.ds((_NB - 1) * _R, _R), :])
            m = (acc_s[0:1, :] + cs) * (1.0 / B)
            v = (acc_s[1:2, :] + cq) * (1.0 / B) - m * m
            s = g0_ref[:] * jax.lax.rsqrt(v + 1e-5)
            w2_s[:, :] = (we_ref[:] * s.reshape(HID, 1)).astype(jnp.bfloat16)
            shift = b0_ref[:] - m * s
            b2_s[0:1, :] = be_ref[:] + jnp.dot(
                shift, we_ref[:],
                preferred_element_type=jnp.float32,
                precision=jax.lax.Precision.HIGHEST)

        o = jnp.dot(h_s[pl.ds(i * _R, _R), :], w2_s[:, :],
                    preferred_element_type=jnp.float32)
        out_ref[:] = jnp.maximum(o + b2_s[0:1, :], 0.0)


def kernel(x, W_start, b_start, bn0_g, bn0_b, W_exp, b_exp, bn_g, bn_b,
           W_end, b_end, W_dqn, b_dqn):
    del W_exp, b_exp, bn_g, bn_b, W_dqn, b_dqn

    row = lambda a: a.reshape(1, -1)
    grid = (2, _NB)
    out = pl.pallas_call(
        _body,
        grid=grid,
        in_specs=[
            pl.BlockSpec((_R, IN_DIMS), lambda p, i: (i * (1 - p), 0)),
            pl.BlockSpec((IN_DIMS, HID), lambda p, i: (0, 0)),
            pl.BlockSpec((1, HID), lambda p, i: (0, 0)),
            pl.BlockSpec((1, HID), lambda p, i: (0, 0)),
            pl.BlockSpec((1, HID), lambda p, i: (0, 0)),
            pl.BlockSpec((HID, OUT), lambda p, i: (0, 0)),
            pl.BlockSpec((1, OUT), lambda p, i: (0, 0)),
        ],
        out_specs=pl.BlockSpec((_R, OUT), lambda p, i: (i * p, 0)),
        out_shape=jax.ShapeDtypeStruct((B, OUT), jnp.float32),
        scratch_shapes=[
            pltpu.VMEM((B, HID), jnp.bfloat16),
            pltpu.VMEM((2, HID), jnp.float32),
            pltpu.VMEM((HID, OUT), jnp.bfloat16),
            pltpu.VMEM((1, OUT), jnp.float32),
        ],
        compiler_params=pltpu.CompilerParams(
            dimension_semantics=("arbitrary", "arbitrary"),
        ),
    )(x, W_start, row(b_start), row(bn0_g), row(bn0_b), W_end, row(b_end))
    return out
